# Initial kernel scaffold; baseline (speedup 1.0000x reference)
import jax
import jax.numpy as jnp
from jax.experimental import pallas as pl


def kernel(input, labels):
    raise NotImplementedError("write your pallas kernel here")


# speedup vs baseline: 3.2350x; 3.2350x over previous
import functools

import jax
import jax.numpy as jnp
from jax.experimental import pallas as pl
from jax.experimental.pallas import tpu as pltpu

S = 20.0
M = 0.2


def _body(inp_ref, lab_ref, out_ref, m_s, s_s, t_s, loss_s, *, C, Rb, Cb, B):
    i = pl.program_id(0)
    j = pl.program_id(1)
    nr = pl.num_programs(0)
    nc = pl.num_programs(1)

    @pl.when(j == 0)
    def _():
        m_s[...] = jnp.full((Rb, 1), -jnp.inf, jnp.float32)
        s_s[...] = jnp.zeros((Rb, 1), jnp.float32)
        t_s[...] = jnp.zeros((Rb, 1), jnp.float32)

    @pl.when((i == 0) & (j == 0))
    def _():
        loss_s[0] = 0.0

    cos = inp_ref[...]
    col = j * Cb + jax.lax.broadcasted_iota(jnp.int32, (Rb, Cb), 1)
    lab = lab_ref[...]
    islab = col == lab
    x = jnp.where(islab, S * cos - S * M, S * cos)
    x = jnp.where(col < C, x, -jnp.inf)
    t_s[...] += jnp.sum(jnp.where(islab, cos, 0.0), axis=1, keepdims=True)
    mloc = jnp.max(x, axis=1, keepdims=True)
    mold = m_s[...]
    mnew = jnp.maximum(mold, mloc)
    m_s[...] = mnew
    s_s[...] = s_s[...] * jnp.exp(mold - mnew) + jnp.sum(
        jnp.exp(x - mnew), axis=1, keepdims=True
    )

    @pl.when(j == nc - 1)
    def _():
        lse = m_s[...] + jnp.log(s_s[...])
        tgt = S * (t_s[...] - M)
        loss_s[0] += jnp.sum(lse - tgt)

    @pl.when((i == nr - 1) & (j == nc - 1))
    def _():
        out_ref[0] = loss_s[0] / B


@jax.jit
def kernel(input, labels):
    B, C = input.shape
    lab = labels.reshape(B, 1).astype(jnp.int32)
    Rb = 2048
    Cb = 1024
    nr = B // Rb
    nc = pl.cdiv(C, Cb)
    out = pl.pallas_call(
        functools.partial(_body, C=C, Rb=Rb, Cb=Cb, B=B),
        grid=(nr, nc),
        in_specs=[
            pl.BlockSpec((Rb, Cb), lambda i, j: (i, j)),
            pl.BlockSpec((Rb, 1), lambda i, j: (i, 0)),
        ],
        out_specs=pl.BlockSpec(memory_space=pltpu.SMEM),
        out_shape=jax.ShapeDtypeStruct((1,), jnp.float32),
        scratch_shapes=[
            pltpu.VMEM((Rb, 1), jnp.float32),
            pltpu.VMEM((Rb, 1), jnp.float32),
            pltpu.VMEM((Rb, 1), jnp.float32),
            pltpu.SMEM((1,), jnp.float32),
        ],
    )(input, lab)
    return out[0]
